# Initial kernel scaffold; baseline (speedup 1.0000x reference)
import functools

import jax
import jax.numpy as jnp
from jax import lax
from jax.experimental import pallas as pl
from jax.experimental.pallas import tpu as pltpu
from jax.experimental.pallas import tpu_sc as plsc

N_CORES = 2
N_SUBCORES = 16
LANES = 16
NW = N_CORES * N_SUBCORES
K_EDGES = 128


def _sc_aggregate(x, src_r, dst_r, w_r, n_nodes):
    cpt = src_r.shape[1]
    feat = x.shape[1]
    n_fg = feat // LANES
    rows_per_tile = n_nodes // N_SUBCORES

    @functools.partial(
        pl.kernel,
        out_type=jax.ShapeDtypeStruct((N_CORES, n_nodes, feat), jnp.float32),
        mesh=plsc.VectorSubcoreMesh(core_axis_name="c", subcore_axis_name="s"),
        scratch_types=[
            pltpu.VMEM((K_EDGES,), jnp.int32),
            pltpu.VMEM((K_EDGES,), jnp.int32),
            pltpu.VMEM((K_EDGES,), jnp.float32),
            pltpu.VMEM((K_EDGES, feat), jnp.float32),
            pltpu.VMEM_SHARED((n_nodes, feat), jnp.float32),
            pltpu.SemaphoreType.DMA,
        ],
    )
    def sc_kernel(x_hbm, src_hbm, dst_hbm, w_hbm, out_hbm,
                  src_v, dst_v, w_v, rows_v, agg, sem):
        c = lax.axis_index("c")
        s = lax.axis_index("s")
        wid = c * N_SUBCORES + s

        def zero_row(r, carry):
            for j in range(n_fg):
                rows_v[r, pl.ds(j * LANES, LANES)] = jnp.zeros(
                    (LANES,), jnp.float32)
            return carry
        lax.fori_loop(0, K_EDGES, zero_row, 0)

        base = s * rows_per_tile
        n_full = rows_per_tile // K_EDGES
        rem = rows_per_tile % K_EDGES
        for t in range(n_full):
            pltpu.sync_copy(rows_v, agg.at[pl.ds(base + t * K_EDGES, K_EDGES)])
        if rem:
            pltpu.sync_copy(rows_v.at[pl.ds(0, rem)],
                            agg.at[pl.ds(base + n_full * K_EDGES, rem)])
        plsc.subcore_barrier()

        def chunk_body(ci, carry):
            pltpu.sync_copy(src_hbm.at[wid, ci], src_v)
            pltpu.sync_copy(dst_hbm.at[wid, ci], dst_v)
            pltpu.sync_copy(w_hbm.at[wid, ci], w_v)
            pltpu.async_copy(x_hbm.at[src_v], rows_v, sem).wait()

            def row_body(e, carry2):
                wb = plsc.load_gather(w_v, [jnp.full((LANES,), e, jnp.int32)])
                for j in range(n_fg):
                    sl = pl.ds(j * LANES, LANES)
                    rows_v[e, sl] = rows_v[e, sl] * wb
                return carry2
            lax.fori_loop(0, K_EDGES, row_body, 0)

            pltpu.sync_copy(rows_v, agg.at[dst_v], add=True)
            return carry
        lax.fori_loop(0, cpt, chunk_body, 0)

        plsc.subcore_barrier()
        pltpu.sync_copy(agg.at[pl.ds(base, rows_per_tile)],
                        out_hbm.at[c, pl.ds(base, rows_per_tile)])

    return sc_kernel(x, src_r, dst_r, w_r)


def _project(parts, W, b):
    m = parts.shape[1]
    feat = parts.shape[2]
    bm = 500

    def mm_kernel(p_ref, w_ref, b_ref, o_ref):
        acc = p_ref[0] + p_ref[1]
        o_ref[...] = jnp.dot(acc, w_ref[...],
                             preferred_element_type=jnp.float32) \
            + b_ref[...][None, :]

    return pl.pallas_call(
        mm_kernel,
        grid=(m // bm,),
        in_specs=[
            pl.BlockSpec((N_CORES, bm, feat), lambda i: (0, i, 0)),
            pl.BlockSpec((feat, feat), lambda i: (0, 0)),
            pl.BlockSpec((feat,), lambda i: (0,)),
        ],
        out_specs=pl.BlockSpec((bm, feat), lambda i: (i, 0)),
        out_shape=jax.ShapeDtypeStruct((m, feat), jnp.float32),
    )(parts, W, b)


def kernel(x, edge_index, edge_weight, W, b):
    n_nodes = x.shape[0]
    n_edges = edge_weight.shape[0]
    cpt = -(-n_edges // (NW * K_EDGES))
    padded = NW * cpt * K_EDGES
    pad = padded - n_edges

    dst = edge_index[0]
    src = edge_index[1]
    ew = edge_weight
    if pad:
        dst = jnp.concatenate([dst, jnp.zeros((pad,), dst.dtype)])
        src = jnp.concatenate([src, jnp.zeros((pad,), src.dtype)])
        ew = jnp.concatenate([ew, jnp.zeros((pad,), ew.dtype)])

    src_r = src.reshape(NW, cpt, K_EDGES)
    dst_r = dst.reshape(NW, cpt, K_EDGES)
    w_r = ew.reshape(NW, cpt, K_EDGES)

    parts = _sc_aggregate(x, src_r, dst_r, w_r, n_nodes)
    return _project(parts, W, b)


# speedup vs baseline: 3.6703x; 3.6703x over previous
import functools

import jax
import jax.numpy as jnp
from jax import lax
from jax.experimental import pallas as pl
from jax.experimental.pallas import tpu as pltpu
from jax.experimental.pallas import tpu_sc as plsc

N_CORES = 2
N_SUBCORES = 16
LANES = 16
NW = N_CORES * N_SUBCORES
K_EDGES = 128


def _lane_broadcast(v, lane):
    idx = jnp.full((LANES,), lane, jnp.int32)
    return lax.gather(
        v, idx[:, None],
        lax.GatherDimensionNumbers(
            offset_dims=(), collapsed_slice_dims=(0,), start_index_map=(0,)),
        (1,), mode=lax.GatherScatterMode.PROMISE_IN_BOUNDS)


def _sc_aggregate(x, src_r, dst_r, w_r, n_nodes):
    cpt = src_r.shape[1]
    feat = x.shape[1]
    n_fg = feat // LANES
    n_pad = -(-n_nodes // (N_SUBCORES * K_EDGES)) * N_SUBCORES * K_EDGES
    rows_per_tile = n_pad // N_SUBCORES

    @functools.partial(
        pl.kernel,
        out_type=jax.ShapeDtypeStruct((N_CORES, n_pad, feat), jnp.float32),
        mesh=plsc.VectorSubcoreMesh(core_axis_name="c", subcore_axis_name="s"),
        scratch_types=[
            pltpu.VMEM((K_EDGES,), jnp.int32),
            pltpu.VMEM((K_EDGES,), jnp.int32),
            pltpu.VMEM((K_EDGES,), jnp.float32),
            pltpu.VMEM((K_EDGES, feat), jnp.float32),
            pltpu.VMEM_SHARED((n_pad, feat), jnp.float32),
            pltpu.SemaphoreType.DMA,
        ],
    )
    def sc_kernel(x_hbm, src_hbm, dst_hbm, w_hbm, out_hbm,
                  src_v, dst_v, w_v, rows_v, agg, sem):
        c = lax.axis_index("c")
        s = lax.axis_index("s")
        wid = c * N_SUBCORES + s

        def zero_row(r, carry):
            for j in range(n_fg):
                rows_v[r, pl.ds(j * LANES, LANES)] = jnp.zeros(
                    (LANES,), jnp.float32)
            return carry
        lax.fori_loop(0, K_EDGES, zero_row, 0)

        base = s * rows_per_tile
        n_full = rows_per_tile // K_EDGES
        rem = rows_per_tile % K_EDGES
        for t in range(n_full):
            pltpu.sync_copy(rows_v, agg.at[pl.ds(base + t * K_EDGES, K_EDGES)])
        if rem:
            pltpu.sync_copy(rows_v.at[pl.ds(0, rem)],
                            agg.at[pl.ds(base + n_full * K_EDGES, rem)])
        plsc.subcore_barrier()

        def chunk_body(ci, carry):
            pltpu.sync_copy(src_hbm.at[wid, ci], src_v)
            pltpu.sync_copy(dst_hbm.at[wid, ci], dst_v)
            pltpu.sync_copy(w_hbm.at[wid, ci], w_v)
            pltpu.async_copy(x_hbm.at[src_v], rows_v, sem).wait()

            def group_body(g, carry2):
                wv = w_v[pl.ds(g * LANES, LANES)]
                for l in range(LANES):
                    wb = _lane_broadcast(wv, l)
                    e = g * LANES + l
                    for j in range(n_fg):
                        sl = pl.ds(j * LANES, LANES)
                        rows_v[e, sl] = rows_v[e, sl] * wb
                return carry2
            lax.fori_loop(0, K_EDGES // LANES, group_body, 0)

            pltpu.sync_copy(rows_v, agg.at[dst_v], add=True)
            return carry
        lax.fori_loop(0, cpt, chunk_body, 0)

        plsc.subcore_barrier()
        pltpu.sync_copy(agg.at[pl.ds(base, rows_per_tile)],
                        out_hbm.at[c, pl.ds(base, rows_per_tile)])

    return sc_kernel(x, src_r, dst_r, w_r)


def _project(parts, W, b, m):
    feat = parts.shape[2]
    bm = 1000

    def mm_kernel(p_ref, w_ref, b_ref, o_ref):
        acc = p_ref[0] + p_ref[1]
        o_ref[...] = jnp.dot(acc, w_ref[...],
                             preferred_element_type=jnp.float32) \
            + b_ref[...][None, :]

    return pl.pallas_call(
        mm_kernel,
        grid=(m // bm,),
        in_specs=[
            pl.BlockSpec((N_CORES, bm, feat), lambda i: (0, i, 0)),
            pl.BlockSpec((feat, feat), lambda i: (0, 0)),
            pl.BlockSpec((feat,), lambda i: (0,)),
        ],
        out_specs=pl.BlockSpec((bm, feat), lambda i: (i, 0)),
        out_shape=jax.ShapeDtypeStruct((m, feat), jnp.float32),
    )(parts, W, b)


def kernel(x, edge_index, edge_weight, W, b):
    n_nodes = x.shape[0]
    n_edges = edge_weight.shape[0]
    cpt = -(-n_edges // (NW * K_EDGES))
    padded = NW * cpt * K_EDGES
    pad = padded - n_edges

    dst = edge_index[0]
    src = edge_index[1]
    ew = edge_weight
    if pad:
        dst = jnp.concatenate([dst, jnp.zeros((pad,), dst.dtype)])
        src = jnp.concatenate([src, jnp.zeros((pad,), src.dtype)])
        ew = jnp.concatenate([ew, jnp.zeros((pad,), ew.dtype)])

    src_r = src.reshape(NW, cpt, K_EDGES)
    dst_r = dst.reshape(NW, cpt, K_EDGES)
    w_r = ew.reshape(NW, cpt, K_EDGES)

    parts = _sc_aggregate(x, src_r, dst_r, w_r, n_nodes)
    return _project(parts, W, b, n_nodes)
